# Initial kernel scaffold; baseline (speedup 1.0000x reference)
import jax
import jax.numpy as jnp
from jax.experimental import pallas as pl


def kernel(node_feats, edge_index, edge_feats, W_in, W_cm, b_cm, W_e, W_n, W_s):
    raise NotImplementedError("write your pallas kernel here")


# speedup vs baseline: 2.8428x; 2.8428x over previous
import functools

import jax
import jax.numpy as jnp
from jax import lax
from jax.experimental import pallas as pl
from jax.experimental.pallas import tpu as pltpu
from jax.experimental.pallas import tpu_sc as plsc

V = 10000
E = 320000
D = 128
D_EDGE = 16

NC = 2
NS = 16
NW = NC * NS
EPW = E // NW
CHUNK = 80
NCHUNK = EPW // CHUNK
VPAD = 10240
VPS = VPAD // NS

NODE_BLK = 1000
EDGE_BLK = 4000


def _node_mm_body(x_ref, win_ref, wn_ref, ws_ref, hv_ref, hs_ref):
    h = jnp.maximum(
        jnp.dot(x_ref[...], win_ref[...], preferred_element_type=jnp.float32), 0.0
    )
    hv_ref[...] = jnp.dot(h, wn_ref[...], preferred_element_type=jnp.float32)
    hs_ref[...] = jnp.dot(h, ws_ref[...], preferred_element_type=jnp.float32)


def _node_mm(x, w_in, w_n, w_s):
    return pl.pallas_call(
        _node_mm_body,
        grid=(V // NODE_BLK,),
        in_specs=[
            pl.BlockSpec((NODE_BLK, D), lambda i: (i, 0)),
            pl.BlockSpec((D, D), lambda i: (0, 0)),
            pl.BlockSpec((D, D), lambda i: (0, 0)),
            pl.BlockSpec((D, D), lambda i: (0, 0)),
        ],
        out_specs=[
            pl.BlockSpec((NODE_BLK, D), lambda i: (i, 0)),
            pl.BlockSpec((NODE_BLK, D), lambda i: (i, 0)),
        ],
        out_shape=[
            jax.ShapeDtypeStruct((V, D), jnp.float32),
            jax.ShapeDtypeStruct((V, D), jnp.float32),
        ],
    )(x, w_in, w_n, w_s)


def _edge_mm_body(ef_ref, we_ref, he2_ref):
    he2_ref[...] = jnp.dot(
        ef_ref[...], we_ref[...], preferred_element_type=jnp.float32
    )


def _edge_mm(ef, w_e):
    return pl.pallas_call(
        _edge_mm_body,
        grid=(E // EDGE_BLK,),
        in_specs=[
            pl.BlockSpec((EDGE_BLK, D_EDGE), lambda i: (i, 0)),
            pl.BlockSpec((D_EDGE, D), lambda i: (0, 0)),
        ],
        out_specs=pl.BlockSpec((EDGE_BLK, D), lambda i: (i, 0)),
        out_shape=jax.ShapeDtypeStruct((E, D), jnp.float32),
    )(ef, w_e)


def _edge_sc_body(hv_hbm, src_hbm, dst_hbm, he2_hbm, zeros_hbm, out_hbm,
                  sidx, didx, grows, erows, acc, sem):
    c = lax.axis_index("c")
    s = lax.axis_index("s")
    wid = c * NS + s

    zbase = s * VPS
    pltpu.sync_copy(zeros_hbm.at[pl.ds(zbase, VPS)], acc.at[pl.ds(zbase, VPS)])
    plsc.subcore_barrier()

    ebase = wid * EPW

    @pl.loop(0, NCHUNK)
    def _chunk(g):
        off = ebase + g * CHUNK
        pltpu.sync_copy(src_hbm.at[pl.ds(off, CHUNK)], sidx)
        pltpu.sync_copy(dst_hbm.at[pl.ds(off, CHUNK)], didx)
        gather = pltpu.async_copy(hv_hbm.at[sidx], grows, sem)
        pltpu.sync_copy(he2_hbm.at[pl.ds(off, CHUNK)], erows)
        gather.wait()

        @plsc.parallel_loop(0, CHUNK)
        def _mul(i):
            for j in range(D // 16):
                sl = pl.ds(j * 16, 16)
                grows[i, sl] = grows[i, sl] * erows[i, sl]

        pltpu.sync_copy(grows, acc.at[didx], add=True)

    plsc.subcore_barrier()
    pltpu.sync_copy(acc.at[pl.ds(zbase, VPS)], out_hbm.at[c, pl.ds(zbase, VPS)])


def _edge_sc(hv, src, dst, he2, zeros):
    mesh = plsc.VectorSubcoreMesh(
        core_axis_name="c", subcore_axis_name="s", num_cores=NC, num_subcores=NS
    )
    return pl.kernel(
        _edge_sc_body,
        out_type=jax.ShapeDtypeStruct((NC, VPAD, D), jnp.float32),
        mesh=mesh,
        scratch_types=[
            pltpu.VMEM((CHUNK,), jnp.int32),
            pltpu.VMEM((CHUNK,), jnp.int32),
            pltpu.VMEM((CHUNK, D), jnp.float32),
            pltpu.VMEM((CHUNK, D), jnp.float32),
            pltpu.VMEM_SHARED((VPAD, D), jnp.float32),
            pltpu.SemaphoreType.DMA,
        ],
    )(hv, src, dst, he2, zeros)


def _combine_body(p_ref, hs_ref, out_ref):
    out_ref[...] = (p_ref[0] + p_ref[1]) * hs_ref[...]


def _combine(partials, h_self):
    return pl.pallas_call(
        _combine_body,
        grid=(V // NODE_BLK,),
        in_specs=[
            pl.BlockSpec((NC, NODE_BLK, D), lambda i: (0, i, 0)),
            pl.BlockSpec((NODE_BLK, D), lambda i: (i, 0)),
        ],
        out_specs=pl.BlockSpec((NODE_BLK, D), lambda i: (i, 0)),
        out_shape=jax.ShapeDtypeStruct((V, D), jnp.float32),
    )(partials, h_self)


def kernel(node_feats, edge_index, edge_feats, W_in, W_cm, b_cm, W_e, W_n, W_s):
    src = edge_index[0]
    dst = edge_index[1]
    hv, h_self = _node_mm(node_feats, W_in, W_n, W_s)
    he2 = _edge_mm(edge_feats, W_e)
    zeros = jnp.zeros((VPAD, D), jnp.float32)
    partials = _edge_sc(hv, src, dst, he2, zeros)
    return _combine(partials, h_self)


# speedup vs baseline: 3.8798x; 1.3647x over previous
import functools

import jax
import jax.numpy as jnp
from jax import lax
from jax.experimental import pallas as pl
from jax.experimental.pallas import tpu as pltpu
from jax.experimental.pallas import tpu_sc as plsc

V = 10000
E = 320000
D = 128
D_EDGE = 16

NC = 2
NS = 16
NW = NC * NS
EPW = E // NW
CHUNK = 40
NCHUNK = EPW // CHUNK
NISLOT = 5
UNROLL = 10
VPAD = 10240
VPS = VPAD // NS

NODE_BLK = 1000
EDGE_BLK = 4000


def _node_mm_body(x_ref, win_ref, wn_ref, ws_ref, hv_ref, hs_ref):
    h = jnp.maximum(
        jnp.dot(x_ref[...], win_ref[...], preferred_element_type=jnp.float32), 0.0
    )
    hv_ref[...] = jnp.dot(h, wn_ref[...], preferred_element_type=jnp.float32)
    hs_ref[...] = jnp.dot(h, ws_ref[...], preferred_element_type=jnp.float32)


def _node_mm(x, w_in, w_n, w_s):
    return pl.pallas_call(
        _node_mm_body,
        grid=(V // NODE_BLK,),
        in_specs=[
            pl.BlockSpec((NODE_BLK, D), lambda i: (i, 0)),
            pl.BlockSpec((D, D), lambda i: (0, 0)),
            pl.BlockSpec((D, D), lambda i: (0, 0)),
            pl.BlockSpec((D, D), lambda i: (0, 0)),
        ],
        out_specs=[
            pl.BlockSpec((NODE_BLK, D), lambda i: (i, 0)),
            pl.BlockSpec((NODE_BLK, D), lambda i: (i, 0)),
        ],
        out_shape=[
            jax.ShapeDtypeStruct((V, D), jnp.float32),
            jax.ShapeDtypeStruct((V, D), jnp.float32),
        ],
    )(x, w_in, w_n, w_s)


def _edge_mm_body(ef_ref, we_ref, he2_ref):
    he2_ref[...] = jnp.dot(
        ef_ref[...], we_ref[...], preferred_element_type=jnp.float32
    )


def _edge_mm(ef, w_e):
    return pl.pallas_call(
        _edge_mm_body,
        grid=(E // EDGE_BLK,),
        in_specs=[
            pl.BlockSpec((EDGE_BLK, D_EDGE), lambda i: (i, 0)),
            pl.BlockSpec((D_EDGE, D), lambda i: (0, 0)),
        ],
        out_specs=pl.BlockSpec((EDGE_BLK, D), lambda i: (i, 0)),
        out_shape=jax.ShapeDtypeStruct((E, D), jnp.float32),
    )(ef, w_e)


def _edge_sc_body(hv_hbm, idx_hbm, he2_hbm, zeros_hbm, out_hbm,
                  idxv, grows, erows, acc, si0, si1, si2, si3, si4,
                  sg0, sg1, sh0, sh1):
    c = lax.axis_index("c")
    s = lax.axis_index("s")
    wid = c * NS + s

    zbase = s * VPS
    pltpu.sync_copy(zeros_hbm.at[pl.ds(zbase, VPS)], acc.at[pl.ds(zbase, VPS)])
    plsc.subcore_barrier()

    ebase = wid * EPW
    sis = (si0, si1, si2, si3, si4)
    sgs = (sg0, sg1)
    shs = (sh0, sh1)

    def _issue_idx(g, i):
        pltpu.async_copy(idx_hbm.at[wid, g], idxv.at[i], sis[i])

    def _wait_idx(i):
        pltpu.make_async_copy(idx_hbm.at[0, 0], idxv.at[i], sis[i]).wait()

    def _issue_data(g, i, b):
        pltpu.async_copy(hv_hbm.at[idxv.at[i, 0]], grows.at[b], sgs[b])
        pltpu.async_copy(he2_hbm.at[pl.ds(ebase + g * CHUNK, CHUNK)],
                         erows.at[b], shs[b])

    def _drain_data(b):
        pltpu.make_async_copy(hv_hbm.at[idxv.at[0, 0]], grows.at[b],
                              sgs[b]).wait()
        pltpu.make_async_copy(he2_hbm.at[pl.ds(ebase, CHUNK)], erows.at[b],
                              shs[b]).wait()

    def _mul_scatter(i, b):
        @plsc.parallel_loop(0, CHUNK)
        def _mul(r):
            for j in range(D // 16):
                sl = pl.ds(j * 16, 16)
                grows[b, r, sl] = grows[b, r, sl] * erows[b, r, sl]

        pltpu.sync_copy(grows.at[b], acc.at[idxv.at[i, 1]], add=True)

    def _phase(g, base):
        i, b = g % NISLOT, g % 2
        if g + 1 < NCHUNK:
            _wait_idx((g + 1) % NISLOT)
            _issue_data(base + (g + 1), (g + 1) % NISLOT, (g + 1) % 2)
        _drain_data(b)
        _mul_scatter(i, b)
        if g + 4 < NCHUNK:
            _issue_idx(base + (g + 4), (g + 4) % NISLOT)

    for g in range(4):
        _issue_idx(g, g)
    _wait_idx(0)
    _issue_data(0, 0, 0)

    @pl.loop(0, (NCHUNK - UNROLL) // UNROLL)
    def _block(blk):
        base = blk * UNROLL
        for k in range(UNROLL):
            _phase(k, base)

    for g in range(NCHUNK - UNROLL, NCHUNK):
        _phase(g, 0)

    plsc.subcore_barrier()
    pltpu.sync_copy(acc.at[pl.ds(zbase, VPS)], out_hbm.at[c, pl.ds(zbase, VPS)])


def _edge_sc(hv, idx, he2, zeros):
    mesh = plsc.VectorSubcoreMesh(
        core_axis_name="c", subcore_axis_name="s", num_cores=NC, num_subcores=NS
    )
    return pl.kernel(
        _edge_sc_body,
        out_type=jax.ShapeDtypeStruct((NC, VPAD, D), jnp.float32),
        mesh=mesh,
        scratch_types=[
            pltpu.VMEM((NISLOT, 2, CHUNK), jnp.int32),
            pltpu.VMEM((2, CHUNK, D), jnp.float32),
            pltpu.VMEM((2, CHUNK, D), jnp.float32),
            pltpu.VMEM_SHARED((VPAD, D), jnp.float32),
            pltpu.SemaphoreType.DMA,
            pltpu.SemaphoreType.DMA,
            pltpu.SemaphoreType.DMA,
            pltpu.SemaphoreType.DMA,
            pltpu.SemaphoreType.DMA,
            pltpu.SemaphoreType.DMA,
            pltpu.SemaphoreType.DMA,
            pltpu.SemaphoreType.DMA,
            pltpu.SemaphoreType.DMA,
        ],
    )(hv, idx, he2, zeros)


def _combine_body(p_ref, hs_ref, out_ref):
    out_ref[...] = (p_ref[0] + p_ref[1]) * hs_ref[...]


def _combine(partials, h_self):
    return pl.pallas_call(
        _combine_body,
        grid=(V // NODE_BLK,),
        in_specs=[
            pl.BlockSpec((NC, NODE_BLK, D), lambda i: (0, i, 0)),
            pl.BlockSpec((NODE_BLK, D), lambda i: (i, 0)),
        ],
        out_specs=pl.BlockSpec((NODE_BLK, D), lambda i: (i, 0)),
        out_shape=jax.ShapeDtypeStruct((V, D), jnp.float32),
    )(partials, h_self)


def kernel(node_feats, edge_index, edge_feats, W_in, W_cm, b_cm, W_e, W_n, W_s):
    idx = jnp.stack(
        [edge_index[0].reshape(NW, NCHUNK, CHUNK),
         edge_index[1].reshape(NW, NCHUNK, CHUNK)], axis=2
    )
    hv, h_self = _node_mm(node_feats, W_in, W_n, W_s)
    he2 = _edge_mm(edge_feats, W_e)
    zeros = jnp.zeros((VPAD, D), jnp.float32)
    partials = _edge_sc(hv, idx, he2, zeros)
    return _combine(partials, h_self)


# speedup vs baseline: 3.8813x; 1.0004x over previous
import functools

import jax
import jax.numpy as jnp
from jax import lax
from jax.experimental import pallas as pl
from jax.experimental.pallas import tpu as pltpu
from jax.experimental.pallas import tpu_sc as plsc

V = 10000
E = 320000
D = 128
D_EDGE = 16

NC = 2
NS = 16
NW = NC * NS
EPW = E // NW
CHUNK = 40
NCHUNK = EPW // CHUNK
NISLOT = 5
UNROLL = 10
VPAD = 10240
VPS = VPAD // NS

NODE_BLK = 1000
EDGE_BLK = 4000


def _node_mm_body(x_ref, win_ref, wn_ref, ws_ref, hv_ref, hs_ref):
    h = jnp.maximum(
        jnp.dot(x_ref[...], win_ref[...], preferred_element_type=jnp.float32), 0.0
    )
    hv_ref[...] = jnp.dot(h, wn_ref[...], preferred_element_type=jnp.float32)
    hs_ref[...] = jnp.dot(h, ws_ref[...], preferred_element_type=jnp.float32)


def _node_mm(x, w_in, w_n, w_s):
    return pl.pallas_call(
        _node_mm_body,
        grid=(V // NODE_BLK,),
        in_specs=[
            pl.BlockSpec((NODE_BLK, D), lambda i: (i, 0)),
            pl.BlockSpec((D, D), lambda i: (0, 0)),
            pl.BlockSpec((D, D), lambda i: (0, 0)),
            pl.BlockSpec((D, D), lambda i: (0, 0)),
        ],
        out_specs=[
            pl.BlockSpec((NODE_BLK, D), lambda i: (i, 0)),
            pl.BlockSpec((NODE_BLK, D), lambda i: (i, 0)),
        ],
        out_shape=[
            jax.ShapeDtypeStruct((V, D), jnp.float32),
            jax.ShapeDtypeStruct((V, D), jnp.float32),
        ],
    )(x, w_in, w_n, w_s)


def _edge_mm_body(ef_ref, wa_ref, wb_ref, he2_ref):
    a = jnp.dot(ef_ref[...], wa_ref[...], preferred_element_type=jnp.float32)
    b = jnp.dot(ef_ref[...], wb_ref[...], preferred_element_type=jnp.float32)
    au = jax.lax.bitcast_convert_type(
        a.astype(jnp.bfloat16), jnp.uint16).astype(jnp.int32)
    bu = jax.lax.bitcast_convert_type(
        b.astype(jnp.bfloat16), jnp.uint16).astype(jnp.int32)
    he2_ref[...] = au | (bu << 16)


def _edge_mm(ef, w_e):
    cols = jnp.arange(D).reshape(D // 32, 2, 16)
    w_a = w_e[:, cols[:, 0, :].reshape(-1)]
    w_b = w_e[:, cols[:, 1, :].reshape(-1)]
    return pl.pallas_call(
        _edge_mm_body,
        grid=(E // EDGE_BLK,),
        in_specs=[
            pl.BlockSpec((EDGE_BLK, D_EDGE), lambda i: (i, 0)),
            pl.BlockSpec((D_EDGE, D // 2), lambda i: (0, 0)),
            pl.BlockSpec((D_EDGE, D // 2), lambda i: (0, 0)),
        ],
        out_specs=pl.BlockSpec((EDGE_BLK, D // 2), lambda i: (i, 0)),
        out_shape=jax.ShapeDtypeStruct((E, D // 2), jnp.int32),
    )(ef, w_a, w_b)


def _edge_sc_body(hv_hbm, idx_hbm, he2_hbm, zeros_hbm, out_hbm,
                  idxv, grows, erows, acc, si0, si1, si2, si3, si4,
                  sg0, sg1, sh0, sh1):
    c = lax.axis_index("c")
    s = lax.axis_index("s")
    wid = c * NS + s

    zbase = s * VPS
    pltpu.sync_copy(zeros_hbm.at[pl.ds(zbase, VPS)], acc.at[pl.ds(zbase, VPS)])
    plsc.subcore_barrier()

    ebase = wid * EPW
    sis = (si0, si1, si2, si3, si4)
    sgs = (sg0, sg1)
    shs = (sh0, sh1)

    def _issue_idx(g, i):
        pltpu.async_copy(idx_hbm.at[wid, g], idxv.at[i], sis[i])

    def _wait_idx(i):
        pltpu.make_async_copy(idx_hbm.at[0, 0], idxv.at[i], sis[i]).wait()

    def _issue_data(g, i, b):
        pltpu.async_copy(hv_hbm.at[idxv.at[i, 0]], grows.at[b], sgs[b])
        pltpu.async_copy(he2_hbm.at[pl.ds(ebase + g * CHUNK, CHUNK)],
                         erows.at[b], shs[b])

    def _drain_data(b):
        pltpu.make_async_copy(hv_hbm.at[idxv.at[0, 0]], grows.at[b],
                              sgs[b]).wait()
        pltpu.make_async_copy(he2_hbm.at[pl.ds(ebase, CHUNK)], erows.at[b],
                              shs[b]).wait()

    def _mul_scatter(i, b):
        @plsc.parallel_loop(0, CHUNK)
        def _mul(r):
            for m in range(D // 32):
                w = erows[b, r, pl.ds(m * 16, 16)]
                af = jax.lax.bitcast_convert_type(w << 16, jnp.float32)
                bf = jax.lax.bitcast_convert_type(
                    w & jnp.int32(-65536), jnp.float32)
                sl0 = pl.ds((2 * m) * 16, 16)
                sl1 = pl.ds((2 * m + 1) * 16, 16)
                grows[b, r, sl0] = grows[b, r, sl0] * af
                grows[b, r, sl1] = grows[b, r, sl1] * bf

        pltpu.sync_copy(grows.at[b], acc.at[idxv.at[i, 1]], add=True)

    def _phase(g, base):
        i, b = g % NISLOT, g % 2
        if g + 1 < NCHUNK:
            _wait_idx((g + 1) % NISLOT)
            _issue_data(base + (g + 1), (g + 1) % NISLOT, (g + 1) % 2)
        _drain_data(b)
        _mul_scatter(i, b)
        if g + 4 < NCHUNK:
            _issue_idx(base + (g + 4), (g + 4) % NISLOT)

    for g in range(4):
        _issue_idx(g, g)
    _wait_idx(0)
    _issue_data(0, 0, 0)

    @pl.loop(0, (NCHUNK - UNROLL) // UNROLL)
    def _block(blk):
        base = blk * UNROLL
        for k in range(UNROLL):
            _phase(k, base)

    for g in range(NCHUNK - UNROLL, NCHUNK):
        _phase(g, 0)

    plsc.subcore_barrier()
    pltpu.sync_copy(acc.at[pl.ds(zbase, VPS)], out_hbm.at[c, pl.ds(zbase, VPS)])


def _edge_sc(hv, idx, he2, zeros):
    mesh = plsc.VectorSubcoreMesh(
        core_axis_name="c", subcore_axis_name="s", num_cores=NC, num_subcores=NS
    )
    return pl.kernel(
        _edge_sc_body,
        out_type=jax.ShapeDtypeStruct((NC, VPAD, D), jnp.float32),
        mesh=mesh,
        scratch_types=[
            pltpu.VMEM((NISLOT, 2, CHUNK), jnp.int32),
            pltpu.VMEM((2, CHUNK, D), jnp.float32),
            pltpu.VMEM((2, CHUNK, D // 2), jnp.int32),
            pltpu.VMEM_SHARED((VPAD, D), jnp.float32),
            pltpu.SemaphoreType.DMA,
            pltpu.SemaphoreType.DMA,
            pltpu.SemaphoreType.DMA,
            pltpu.SemaphoreType.DMA,
            pltpu.SemaphoreType.DMA,
            pltpu.SemaphoreType.DMA,
            pltpu.SemaphoreType.DMA,
            pltpu.SemaphoreType.DMA,
            pltpu.SemaphoreType.DMA,
        ],
    )(hv, idx, he2, zeros)


def _combine_body(p_ref, hs_ref, out_ref):
    out_ref[...] = (p_ref[0] + p_ref[1]) * hs_ref[...]


def _combine(partials, h_self):
    return pl.pallas_call(
        _combine_body,
        grid=(V // NODE_BLK,),
        in_specs=[
            pl.BlockSpec((NC, NODE_BLK, D), lambda i: (0, i, 0)),
            pl.BlockSpec((NODE_BLK, D), lambda i: (i, 0)),
        ],
        out_specs=pl.BlockSpec((NODE_BLK, D), lambda i: (i, 0)),
        out_shape=jax.ShapeDtypeStruct((V, D), jnp.float32),
    )(partials, h_self)


def kernel(node_feats, edge_index, edge_feats, W_in, W_cm, b_cm, W_e, W_n, W_s):
    idx = jnp.stack(
        [edge_index[0].reshape(NW, NCHUNK, CHUNK),
         edge_index[1].reshape(NW, NCHUNK, CHUNK)], axis=2
    )
    hv, h_self = _node_mm(node_feats, W_in, W_n, W_s)
    he2 = _edge_mm(edge_feats, W_e)
    zeros = jnp.zeros((VPAD, D), jnp.float32)
    partials = _edge_sc(hv, idx, he2, zeros)
    return _combine(partials, h_self)


# speedup vs baseline: 4.2925x; 1.1059x over previous
import functools

import jax
import jax.numpy as jnp
from jax import lax
from jax.experimental import pallas as pl
from jax.experimental.pallas import tpu as pltpu
from jax.experimental.pallas import tpu_sc as plsc

V = 10000
E = 320000
D = 128
D_EDGE = 16

NC = 2
NS = 16
NW = NC * NS
EPW = E // NW
CHUNK = 80
NCHUNK = EPW // CHUNK
NISLOT = 5
UNROLL = 10
TAIL = NCHUNK % UNROLL
VPAD = 10240
VPS = VPAD // NS

NODE_BLK = 1000
EDGE_BLK = 4000


def _node_mm_body(x_ref, win_ref, wn_ref, ws_ref, hv_ref, hs_ref):
    h = jnp.maximum(
        jnp.dot(x_ref[...], win_ref[...], preferred_element_type=jnp.float32), 0.0
    )
    hv_ref[...] = jnp.dot(h, wn_ref[...], preferred_element_type=jnp.float32)
    hs_ref[...] = jnp.dot(h, ws_ref[...], preferred_element_type=jnp.float32)


def _node_mm(x, w_in, w_n, w_s):
    return pl.pallas_call(
        _node_mm_body,
        grid=(V // NODE_BLK,),
        in_specs=[
            pl.BlockSpec((NODE_BLK, D), lambda i: (i, 0)),
            pl.BlockSpec((D, D), lambda i: (0, 0)),
            pl.BlockSpec((D, D), lambda i: (0, 0)),
            pl.BlockSpec((D, D), lambda i: (0, 0)),
        ],
        out_specs=[
            pl.BlockSpec((NODE_BLK, D), lambda i: (i, 0)),
            pl.BlockSpec((NODE_BLK, D), lambda i: (i, 0)),
        ],
        out_shape=[
            jax.ShapeDtypeStruct((V, D), jnp.float32),
            jax.ShapeDtypeStruct((V, D), jnp.float32),
        ],
    )(x, w_in, w_n, w_s)


def _edge_mm_body(ef2_ref, wlo_ref, whi_ref, he2_ref):
    lo = jnp.dot(ef2_ref[...], wlo_ref[...], preferred_element_type=jnp.float32)
    hi = jnp.dot(ef2_ref[...], whi_ref[...], preferred_element_type=jnp.float32)
    au = jax.lax.bitcast_convert_type(
        lo.astype(jnp.bfloat16), jnp.uint16).astype(jnp.int32)
    bu = jax.lax.bitcast_convert_type(
        hi.astype(jnp.bfloat16), jnp.uint16).astype(jnp.int32)
    he2_ref[...] = au | (bu << 16)


def _edge_mm(ef, w_e):
    cols = jnp.arange(D).reshape(D // 32, 2, 16)
    w_a = w_e[:, cols[:, 0, :].reshape(-1)]
    w_b = w_e[:, cols[:, 1, :].reshape(-1)]
    z = jnp.zeros((D_EDGE, D // 2), jnp.float32)
    w_lo = jnp.concatenate(
        [jnp.concatenate([w_a, z], 1), jnp.concatenate([z, w_a], 1)], 0)
    w_hi = jnp.concatenate(
        [jnp.concatenate([w_b, z], 1), jnp.concatenate([z, w_b], 1)], 0)
    ef2 = ef.reshape(E // 2, 2 * D_EDGE)
    return pl.pallas_call(
        _edge_mm_body,
        grid=(E // EDGE_BLK,),
        in_specs=[
            pl.BlockSpec((EDGE_BLK // 2, 2 * D_EDGE), lambda i: (i, 0)),
            pl.BlockSpec((2 * D_EDGE, D), lambda i: (0, 0)),
            pl.BlockSpec((2 * D_EDGE, D), lambda i: (0, 0)),
        ],
        out_specs=pl.BlockSpec((EDGE_BLK // 2, D), lambda i: (i, 0)),
        out_shape=jax.ShapeDtypeStruct((E // 2, D), jnp.int32),
    )(ef2, w_lo, w_hi)


def _edge_sc_body(hv_hbm, idx_hbm, he2_hbm, zeros_hbm, out_hbm,
                  idxv, grows, erows, acc, si0, si1, si2, si3, si4,
                  sg0, sg1, sh0, sh1):
    c = lax.axis_index("c")
    s = lax.axis_index("s")
    wid = c * NS + s

    zbase = s * VPS
    pltpu.sync_copy(zeros_hbm.at[pl.ds(zbase, VPS)], acc.at[pl.ds(zbase, VPS)])
    plsc.subcore_barrier()

    ebase = wid * EPW
    ebaseh = wid * (EPW // 2)
    sis = (si0, si1, si2, si3, si4)
    sgs = (sg0, sg1)
    shs = (sh0, sh1)

    def _issue_idx(g, i):
        pltpu.async_copy(idx_hbm.at[wid, g], idxv.at[i], sis[i])

    def _wait_idx(i):
        pltpu.make_async_copy(idx_hbm.at[0, 0], idxv.at[i], sis[i]).wait()

    def _issue_data(g, i, b):
        pltpu.async_copy(hv_hbm.at[idxv.at[i, 0]], grows.at[b], sgs[b])
        pltpu.async_copy(
            he2_hbm.at[pl.ds(ebaseh + g * (CHUNK // 2), CHUNK // 2)],
            erows.at[b], shs[b])

    def _drain_data(b):
        pltpu.make_async_copy(hv_hbm.at[idxv.at[0, 0]], grows.at[b],
                              sgs[b]).wait()
        pltpu.make_async_copy(
            he2_hbm.at[pl.ds(ebaseh, CHUNK // 2)], erows.at[b],
            shs[b]).wait()

    def _mul_scatter(i, b):
        @plsc.parallel_loop(0, CHUNK // 2)
        def _mul(rp):
            for h in range(2):
                r = 2 * rp + h
                for m in range(D // 32):
                    w = erows[b, rp, pl.ds(h * 64 + m * 16, 16)]
                    af = jax.lax.bitcast_convert_type(w << 16, jnp.float32)
                    bf = jax.lax.bitcast_convert_type(
                        w & jnp.int32(-65536), jnp.float32)
                    sl0 = pl.ds((2 * m) * 16, 16)
                    sl1 = pl.ds((2 * m + 1) * 16, 16)
                    grows[b, r, sl0] = grows[b, r, sl0] * af
                    grows[b, r, sl1] = grows[b, r, sl1] * bf

        pltpu.sync_copy(grows.at[b], acc.at[idxv.at[i, 1]], add=True)

    def _phase(g, base):
        i, b = g % NISLOT, g % 2
        if g + 1 < NCHUNK:
            _wait_idx((g + 1) % NISLOT)
            _issue_data(base + (g + 1), (g + 1) % NISLOT, (g + 1) % 2)
        _drain_data(b)
        _mul_scatter(i, b)
        if g + 4 < NCHUNK:
            _issue_idx(base + (g + 4), (g + 4) % NISLOT)

    for g in range(4):
        _issue_idx(g, g)
    _wait_idx(0)
    _issue_data(0, 0, 0)

    @pl.loop(0, (NCHUNK - TAIL) // UNROLL)
    def _block(blk):
        base = blk * UNROLL
        for k in range(UNROLL):
            _phase(k, base)

    for g in range(NCHUNK - TAIL, NCHUNK):
        _phase(g, 0)

    plsc.subcore_barrier()
    pltpu.sync_copy(acc.at[pl.ds(zbase, VPS)], out_hbm.at[c, pl.ds(zbase, VPS)])


def _edge_sc(hv, idx, he2, zeros):
    mesh = plsc.VectorSubcoreMesh(
        core_axis_name="c", subcore_axis_name="s", num_cores=NC, num_subcores=NS
    )
    return pl.kernel(
        _edge_sc_body,
        out_type=jax.ShapeDtypeStruct((NC, VPAD, D), jnp.float32),
        mesh=mesh,
        scratch_types=[
            pltpu.VMEM((NISLOT, 2, CHUNK), jnp.int32),
            pltpu.VMEM((2, CHUNK, D), jnp.float32),
            pltpu.VMEM((2, CHUNK // 2, D), jnp.int32),
            pltpu.VMEM_SHARED((VPAD, D), jnp.float32),
            pltpu.SemaphoreType.DMA,
            pltpu.SemaphoreType.DMA,
            pltpu.SemaphoreType.DMA,
            pltpu.SemaphoreType.DMA,
            pltpu.SemaphoreType.DMA,
            pltpu.SemaphoreType.DMA,
            pltpu.SemaphoreType.DMA,
            pltpu.SemaphoreType.DMA,
            pltpu.SemaphoreType.DMA,
        ],
    )(hv, idx, he2, zeros)


def _combine_body(p_ref, hs_ref, out_ref):
    out_ref[...] = (p_ref[0] + p_ref[1]) * hs_ref[...]


def _combine(partials, h_self):
    return pl.pallas_call(
        _combine_body,
        grid=(V // NODE_BLK,),
        in_specs=[
            pl.BlockSpec((NC, NODE_BLK, D), lambda i: (0, i, 0)),
            pl.BlockSpec((NODE_BLK, D), lambda i: (i, 0)),
        ],
        out_specs=pl.BlockSpec((NODE_BLK, D), lambda i: (i, 0)),
        out_shape=jax.ShapeDtypeStruct((V, D), jnp.float32),
    )(partials, h_self)


def kernel(node_feats, edge_index, edge_feats, W_in, W_cm, b_cm, W_e, W_n, W_s):
    idx = jnp.stack(
        [edge_index[0].reshape(NW, NCHUNK, CHUNK),
         edge_index[1].reshape(NW, NCHUNK, CHUNK)], axis=2
    )
    hv, h_self = _node_mm(node_feats, W_in, W_n, W_s)
    he2 = _edge_mm(edge_feats, W_e)
    zeros = jnp.zeros((VPAD, D), jnp.float32)
    partials = _edge_sc(hv, idx, he2, zeros)
    return _combine(partials, h_self)
